# baseline (speedup 1.0000x reference)
import functools

import jax
import jax.numpy as jnp
from jax import lax
from jax.experimental import pallas as pl
from jax.experimental.pallas import tpu as pltpu
from jax.experimental.pallas import tpu_sc as plsc

_N = 10000
_E = 320000
_D = 128
_HD = _D // 2
_OUT = 4
_NPAD = 10240
_DUMP = _N
_NC = 2
_NS = 16
_L = 16
_NW = _NC * _NS
_CH = 80
_NB = 5
_NB1 = 6
_RPT = _NPAD // _NS

_NCH_A = _E // _NW // _CH
_NCH_S = _E // _NS // _CH

_SC_PARAMS = pltpu.CompilerParams(use_tc_tiling_on_sc=False)
_MESH = dict(core_axis_name="c", subcore_axis_name="s")


def _zero_fill(buf, n_rows, width):
    @pl.loop(0, n_rows)
    def _(i):
        for j in range(width // _L):
            buf[i, pl.ds(j * _L, _L)] = jnp.zeros((_L,), jnp.float32)


def _ring(table_hbm, acc, srcb, dstb, rows, gsem, ssem, n_chunks, nb,
          const_rows=False):
    def g_start(ch, b):
        pltpu.async_copy(table_hbm.at[srcb.at[ch]], rows[b], gsem[b])

    def g_wait(ch, b):
        pltpu.make_async_copy(
            table_hbm.at[srcb.at[ch]], rows[b], gsem[b]).wait()

    def s_start(ch, b):
        rb = rows[0] if const_rows else rows[b]
        pltpu.async_copy(rb, acc.at[dstb.at[ch]], ssem[b], add=True)

    def s_wait(ch, b):
        rb = rows[0] if const_rows else rows[b]
        pltpu.make_async_copy(rb, acc.at[dstb.at[ch]], ssem[b]).wait()

    n_out = n_chunks // nb
    rem = n_chunks - n_out * nb
    if const_rows:
        @pl.loop(0, n_out)
        def _(t):
            base = t * nb
            for b in range(nb):
                s_start(base + b, b)
            for b in range(nb):
                s_wait(base + b, b)
        for b in range(rem):
            s_start(n_out * nb + b, b)
        for b in range(rem):
            s_wait(n_out * nb + b, b)
        return

    for b in range(nb):
        g_start(b, b)

    @pl.loop(0, n_out - 1)
    def _(t):
        base = t * nb
        for b in range(nb):
            g_wait(base + b, b)
            s_start(base + b, b)
        for b in range(nb):
            s_wait(base + b, b)
            g_start(base + nb + b, b)

    tail = (n_out - 1) * nb
    for b in range(nb):
        g_wait(tail + b, b)
        s_start(tail + b, b)
    for b in range(nb):
        s_wait(tail + b, b)
        if b < rem:
            g_start(tail + nb + b, b)
    for b in range(rem):
        g_wait(tail + nb + b, b)
        s_start(tail + nb + b, b)
    for b in range(rem):
        s_wait(tail + nb + b, b)


def _writeback(c, cbuf, out0, out1, sl):
    @pl.when(c == 0)
    def _():
        pltpu.sync_copy(cbuf, out0.at[sl])

    @pl.when(c != 0)
    def _():
        pltpu.sync_copy(cbuf, out1.at[sl])


def _edge_pass16(do_gather, nb=12):
    scratch = (
        [
            pltpu.VMEM((_NCH_A, _CH), jnp.int32),
            pltpu.VMEM((_NCH_A, _CH), jnp.int32),
            pltpu.VMEM((_RPT, 16), jnp.float32),
            pltpu.VMEM_SHARED((_NPAD, 16), jnp.float32),
        ]
        + [pltpu.VMEM((_CH, 16), jnp.float32) for _ in range(nb)]
        + [pltpu.SemaphoreType.DMA for _ in range(2 * nb)]
    )
    part = jax.ShapeDtypeStruct((_NPAD, 16), jnp.float32)
    if do_gather:
        out_type = (part, part)
    else:
        out_type = (part, part,
                    jax.ShapeDtypeStruct((_NW * _NCH_A, _CH), jnp.int32))

    @functools.partial(
        pl.kernel, mesh=plsc.VectorSubcoreMesh(**_MESH),
        out_type=out_type, scratch_types=scratch,
        compiler_params=_SC_PARAMS,
    )
    def k(*refs):
        if do_gather:
            (src_hbm, dstp_hbm, table_hbm, out0, out1), rest = refs[:5], refs[5:]
        else:
            (src_hbm, dst_hbm, out0, out1, dstp_hbm), rest = refs[:5], refs[5:]
            table_hbm = None
        srcb, dstb, cbuf, acc = rest[:4]
        rows = rest[4:4 + nb]
        gsem = rest[4 + nb:4 + 2 * nb]
        ssem = rest[4 + 2 * nb:]

        c = lax.axis_index("c")
        s = lax.axis_index("s")
        w = c * _NS + s
        r0 = s * _RPT

        if do_gather:
            @pl.when(c == 0)
            def _():
                sl = pl.ds(r0, _RPT)
                pltpu.sync_copy(table_hbm.at[sl], cbuf)
                pltpu.sync_copy(cbuf, acc.at[sl])

            @pl.when(c != 0)
            def _():
                _zero_fill(cbuf, _RPT, 16)
                pltpu.sync_copy(cbuf, acc.at[pl.ds(r0, _RPT)])
        else:
            _zero_fill(cbuf, _RPT, 16)
            pltpu.sync_copy(cbuf, acc.at[pl.ds(r0, _RPT)])

        pltpu.sync_copy(src_hbm.at[pl.ds(w * _NCH_A, _NCH_A)], srcb)
        if do_gather:
            pltpu.sync_copy(dstp_hbm.at[pl.ds(w * _NCH_A, _NCH_A)], dstb)
        else:
            pltpu.sync_copy(dst_hbm.at[pl.ds(w * _NCH_A, _NCH_A)], dstb)

            @pl.loop(0, _NCH_A)
            def _(i):
                for j in range(_CH // _L):
                    sl = pl.ds(j * _L, _L)
                    sv = srcb[i, sl]
                    dv = dstb[i, sl]
                    dstb[i, sl] = jnp.where(
                        sv != dv, dv, jnp.full((_L,), _DUMP, jnp.int32))

            pltpu.sync_copy(dstb, dstp_hbm.at[pl.ds(w * _NCH_A, _NCH_A)])

            @pl.loop(0, _CH)
            def _(j):
                rows[0][j, :] = jnp.full((16,), 1.0, jnp.float32)

        plsc.subcore_barrier()
        _ring(table_hbm, acc, srcb, dstb, rows, gsem, ssem, _NCH_A, nb,
              const_rows=not do_gather)
        plsc.subcore_barrier()

        pltpu.sync_copy(acc.at[pl.ds(r0, _RPT)], cbuf)
        _writeback(c, cbuf, out0, out1, pl.ds(r0, _RPT))

    return k


def _l1_pass():
    cb = 128
    scratch = (
        [
            pltpu.VMEM((_NCH_S, _CH), jnp.int32),
            pltpu.VMEM((_NCH_S, _CH), jnp.int32),
            pltpu.VMEM((cb, _HD), jnp.float32),
            pltpu.VMEM_SHARED((_NPAD, _HD), jnp.float32),
        ]
        + [pltpu.VMEM((_CH, _HD), jnp.float32) for _ in range(_NB1)]
        + [pltpu.SemaphoreType.DMA for _ in range(2 * _NB1)]
    )
    half = jax.ShapeDtypeStruct((_NPAD, _HD), jnp.float32)

    @functools.partial(
        pl.kernel, mesh=plsc.VectorSubcoreMesh(**_MESH),
        out_type=(half, half), scratch_types=scratch,
        compiler_params=_SC_PARAMS,
    )
    def k(src_hbm, dstp_hbm, table_hbm, out0, out1,
          srcb, dstb, cbuf, acc, *rs):
        rows = rs[:_NB1]
        gsem = rs[_NB1:2 * _NB1]
        ssem = rs[2 * _NB1:]

        c = lax.axis_index("c")
        s = lax.axis_index("s")
        r0 = s * _RPT

        for i in range(_RPT // cb):
            sl_t = pl.ds(c * _NPAD + r0 + i * cb, cb)
            sl_a = pl.ds(r0 + i * cb, cb)
            pltpu.sync_copy(table_hbm.at[sl_t], cbuf)
            pltpu.sync_copy(cbuf, acc.at[sl_a])

        pltpu.sync_copy(src_hbm.at[pl.ds(s * _NCH_S, _NCH_S)], srcb)
        pltpu.sync_copy(dstp_hbm.at[pl.ds(s * _NCH_S, _NCH_S)], dstb)
        off = c * _NPAD

        @pl.loop(0, _NCH_S)
        def _(i):
            for j in range(_CH // _L):
                sl = pl.ds(j * _L, _L)
                srcb[i, sl] = srcb[i, sl] + off

        plsc.subcore_barrier()
        _ring(table_hbm, acc, srcb, dstb, rows, gsem, ssem, _NCH_S, _NB1)
        plsc.subcore_barrier()

        for i in range(_RPT // cb):
            sl_a = pl.ds(r0 + i * cb, cb)
            pltpu.sync_copy(acc.at[sl_a], cbuf)
            _writeback(c, cbuf, out0, out1, sl_a)

    return k


_RB = 1280


def _dis_of(da0, da1):
    deg = da0[:, 0] + da1[:, 0] + 1.0
    return lax.rsqrt(deg)


def _b1_body(da0_ref, da1_ref, x_ref, out_ref):
    c = pl.program_id(0)
    dis = _dis_of(da0_ref[...], da1_ref[...])
    xp = x_ref[...] * dis[:, None]
    out_ref[...] = jnp.where(c == 0, xp[:, :_HD], xp[:, _HD:])


def _b2_body(aggl_ref, aggr_ref, da0_ref, da1_ref, w1_ref, b1_ref, w2_ref,
             out_ref):
    dis = _dis_of(da0_ref[...], da1_ref[...])
    t = jnp.concatenate([aggl_ref[...], aggr_ref[...]], axis=1) * dis[:, None]
    h = jnp.dot(t, w1_ref[...], preferred_element_type=jnp.float32)
    h = jnp.maximum(h + b1_ref[...], 0.0)
    g = jnp.dot(h, w2_ref[...], preferred_element_type=jnp.float32)
    out_ref[...] = g * dis[:, None]


def _b3_body(p0_ref, p1_ref, da0_ref, da1_ref, b2_ref, out_ref):
    dis = _dis_of(da0_ref[...], da1_ref[...])
    r = (p0_ref[...] + p1_ref[...]) * dis[:, None] + b2_ref[...]
    out_ref[...] = r[:, :_OUT]


def kernel(x, edge_index, W1, b1, W2, b2):
    src2d = edge_index[0].reshape(_NW * _NCH_A, _CH)
    dst2d = edge_index[1].reshape(_NW * _NCH_A, _CH)
    xpad = jnp.pad(x, ((0, _NPAD - _N), (0, 0)))
    b1r = b1.reshape(1, _D)
    W2p = jnp.pad(W2, ((0, 0), (0, 16 - _OUT)))
    b2p = jnp.pad(b2, (0, 16 - _OUT)).reshape(1, 16)

    da0, da1, dstp2d = _edge_pass16(False)(src2d, dst2d)

    tab1 = pl.pallas_call(
        _b1_body,
        grid=(_NC, _NPAD // _RB),
        in_specs=[
            pl.BlockSpec((_RB, 16), lambda c, i: (i, 0)),
            pl.BlockSpec((_RB, 16), lambda c, i: (i, 0)),
            pl.BlockSpec((_RB, _D), lambda c, i: (i, 0)),
        ],
        out_specs=pl.BlockSpec(
            (_RB, _HD), lambda c, i: (c * (_NPAD // _RB) + i, 0)),
        out_shape=jax.ShapeDtypeStruct((_NC * _NPAD, _HD), jnp.float32),
    )(da0, da1, xpad)

    aggl, aggr = _l1_pass()(src2d, dstp2d, tab1)

    tab2 = pl.pallas_call(
        _b2_body,
        grid=(_NPAD // _RB,),
        in_specs=[
            pl.BlockSpec((_RB, _HD), lambda i: (i, 0)),
            pl.BlockSpec((_RB, _HD), lambda i: (i, 0)),
            pl.BlockSpec((_RB, 16), lambda i: (i, 0)),
            pl.BlockSpec((_RB, 16), lambda i: (i, 0)),
            pl.BlockSpec((_D, _D), lambda i: (0, 0)),
            pl.BlockSpec((1, _D), lambda i: (0, 0)),
            pl.BlockSpec((_D, 16), lambda i: (0, 0)),
        ],
        out_specs=pl.BlockSpec((_RB, 16), lambda i: (i, 0)),
        out_shape=jax.ShapeDtypeStruct((_NPAD, 16), jnp.float32),
    )(aggl, aggr, da0, da1, W1, b1r, W2p)

    p0, p1 = _edge_pass16(True)(src2d, dstp2d, tab2)

    _RO = 2000
    return pl.pallas_call(
        _b3_body,
        grid=(_N // _RO,),
        in_specs=[
            pl.BlockSpec((_RO, 16), lambda i: (i, 0)),
            pl.BlockSpec((_RO, 16), lambda i: (i, 0)),
            pl.BlockSpec((_RO, 16), lambda i: (i, 0)),
            pl.BlockSpec((_RO, 16), lambda i: (i, 0)),
            pl.BlockSpec((1, 16), lambda i: (0, 0)),
        ],
        out_specs=pl.BlockSpec((_RO, _OUT), lambda i: (i, 0)),
        out_shape=jax.ShapeDtypeStruct((_N, _OUT), jnp.float32),
    )(p0, p1, da0, da1, b2p)


# speedup vs baseline: 1.0163x; 1.0163x over previous
import functools

import jax
import jax.numpy as jnp
from jax import lax
from jax.experimental import pallas as pl
from jax.experimental.pallas import tpu as pltpu
from jax.experimental.pallas import tpu_sc as plsc

_N = 10000
_E = 320000
_D = 128
_HD = _D // 2
_OUT = 4
_NPAD = 10240
_DUMP = _N
_NC = 2
_NS = 16
_L = 16
_NW = _NC * _NS
_CH = 80
_NB = 5
_NB1 = 6
_RPT = _NPAD // _NS

_NCH_A = _E // _NW // _CH
_NCH_S = _E // _NS // _CH

_SC_PARAMS = pltpu.CompilerParams(use_tc_tiling_on_sc=False)
_MESH = dict(core_axis_name="c", subcore_axis_name="s")


def _zero_fill(buf, n_rows, width):
    @pl.loop(0, n_rows)
    def _(i):
        for j in range(width // _L):
            buf[i, pl.ds(j * _L, _L)] = jnp.zeros((_L,), jnp.float32)


def _ring(table_hbm, acc, srcb, dstb, rows, gsem, ssem, n_chunks, nb,
          const_rows=False):
    def g_start(ch, b):
        pltpu.async_copy(table_hbm.at[srcb.at[ch]], rows[b], gsem[b])

    def g_wait(ch, b):
        pltpu.make_async_copy(
            table_hbm.at[srcb.at[ch]], rows[b], gsem[b]).wait()

    def s_start(ch, b):
        rb = rows[0] if const_rows else rows[b]
        pltpu.async_copy(rb, acc.at[dstb.at[ch]], ssem[b], add=True)

    def s_wait(ch, b):
        rb = rows[0] if const_rows else rows[b]
        pltpu.make_async_copy(rb, acc.at[dstb.at[ch]], ssem[b]).wait()

    n_out = n_chunks // nb
    rem = n_chunks - n_out * nb
    if const_rows:
        @pl.loop(0, n_out)
        def _(t):
            base = t * nb
            for b in range(nb):
                s_start(base + b, b)
            for b in range(nb):
                s_wait(base + b, b)
        for b in range(rem):
            s_start(n_out * nb + b, b)
        for b in range(rem):
            s_wait(n_out * nb + b, b)
        return

    for b in range(nb):
        g_start(b, b)

    @pl.loop(0, n_out - 1)
    def _(t):
        base = t * nb
        for b in range(nb):
            g_wait(base + b, b)
            s_start(base + b, b)
        for b in range(nb):
            s_wait(base + b, b)
            g_start(base + nb + b, b)

    tail = (n_out - 1) * nb
    for b in range(nb):
        g_wait(tail + b, b)
        s_start(tail + b, b)
    for b in range(nb):
        s_wait(tail + b, b)
        if b < rem:
            g_start(tail + nb + b, b)
    for b in range(rem):
        g_wait(tail + nb + b, b)
        s_start(tail + nb + b, b)
    for b in range(rem):
        s_wait(tail + nb + b, b)


def _edge_pass16(do_gather, nb=12):
    scratch = (
        [
            pltpu.VMEM((_NCH_A, _CH), jnp.int32),
            pltpu.VMEM((_NCH_A, _CH), jnp.int32),
            pltpu.VMEM((_RPT, 16), jnp.float32),
            pltpu.VMEM_SHARED((_NPAD, 16), jnp.float32),
        ]
        + [pltpu.VMEM((_CH, 16), jnp.float32) for _ in range(nb)]
        + [pltpu.SemaphoreType.DMA for _ in range(2 * nb)]
    )
    if do_gather:
        out_type = jax.ShapeDtypeStruct((_NC * _NPAD, 16), jnp.float32)
    else:
        out_type = (
            jax.ShapeDtypeStruct((_NC * _NPAD, 16), jnp.float32),
            jax.ShapeDtypeStruct((_NW * _NCH_A, _CH), jnp.int32),
        )

    @functools.partial(
        pl.kernel, mesh=plsc.VectorSubcoreMesh(**_MESH),
        out_type=out_type, scratch_types=scratch,
        compiler_params=_SC_PARAMS,
    )
    def k(*refs):
        if do_gather:
            (src_hbm, dstp_hbm, table_hbm, out_hbm), rest = refs[:4], refs[4:]
        else:
            (src_hbm, dst_hbm, out_hbm, dstp_hbm), rest = refs[:4], refs[4:]
            table_hbm = None
        srcb, dstb, cbuf, acc = rest[:4]
        rows = rest[4:4 + nb]
        gsem = rest[4 + nb:4 + 2 * nb]
        ssem = rest[4 + 2 * nb:]

        c = lax.axis_index("c")
        s = lax.axis_index("s")
        w = c * _NS + s
        r0 = s * _RPT

        if do_gather:
            @pl.when(c == 0)
            def _():
                sl = pl.ds(r0, _RPT)
                pltpu.sync_copy(table_hbm.at[sl], cbuf)
                pltpu.sync_copy(cbuf, acc.at[sl])

            @pl.when(c != 0)
            def _():
                _zero_fill(cbuf, _RPT, 16)
                pltpu.sync_copy(cbuf, acc.at[pl.ds(r0, _RPT)])
        else:
            _zero_fill(cbuf, _RPT, 16)
            pltpu.sync_copy(cbuf, acc.at[pl.ds(r0, _RPT)])

        pltpu.sync_copy(src_hbm.at[pl.ds(w * _NCH_A, _NCH_A)], srcb)
        if do_gather:
            pltpu.sync_copy(dstp_hbm.at[pl.ds(w * _NCH_A, _NCH_A)], dstb)
        else:
            pltpu.sync_copy(dst_hbm.at[pl.ds(w * _NCH_A, _NCH_A)], dstb)

            @pl.loop(0, _NCH_A)
            def _(i):
                for j in range(_CH // _L):
                    sl = pl.ds(j * _L, _L)
                    sv = srcb[i, sl]
                    dv = dstb[i, sl]
                    dstb[i, sl] = jnp.where(
                        sv != dv, dv, jnp.full((_L,), _DUMP, jnp.int32))

            pltpu.sync_copy(dstb, dstp_hbm.at[pl.ds(w * _NCH_A, _NCH_A)])

            @pl.loop(0, _CH)
            def _(j):
                rows[0][j, :] = jnp.full((16,), 1.0, jnp.float32)

        plsc.subcore_barrier()
        _ring(table_hbm, acc, srcb, dstb, rows, gsem, ssem, _NCH_A, nb,
              const_rows=not do_gather)
        plsc.subcore_barrier()

        sl_acc = pl.ds(r0, _RPT)
        sl_out = pl.ds(c * _NPAD + r0, _RPT)
        pltpu.sync_copy(acc.at[sl_acc], cbuf)
        pltpu.sync_copy(cbuf, out_hbm.at[sl_out])

    return k


def _l1_pass():
    cb = 128
    scratch = (
        [
            pltpu.VMEM((_NCH_S, _CH), jnp.int32),
            pltpu.VMEM((_NCH_S, _CH), jnp.int32),
            pltpu.VMEM((cb, _HD), jnp.float32),
            pltpu.VMEM_SHARED((_NPAD, _HD), jnp.float32),
        ]
        + [pltpu.VMEM((_CH, _HD), jnp.float32) for _ in range(_NB1)]
        + [pltpu.SemaphoreType.DMA for _ in range(2 * _NB1)]
    )

    @functools.partial(
        pl.kernel, mesh=plsc.VectorSubcoreMesh(**_MESH),
        out_type=jax.ShapeDtypeStruct((_NC * _NPAD, _HD), jnp.float32),
        scratch_types=scratch,
        compiler_params=_SC_PARAMS,
    )
    def k(src_hbm, dstp_hbm, table_hbm, out_hbm, srcb, dstb, cbuf, acc, *rs):
        rows = rs[:_NB1]
        gsem = rs[_NB1:2 * _NB1]
        ssem = rs[2 * _NB1:]

        c = lax.axis_index("c")
        s = lax.axis_index("s")
        r0 = s * _RPT

        for i in range(_RPT // cb):
            sl_t = pl.ds(c * _NPAD + r0 + i * cb, cb)
            sl_a = pl.ds(r0 + i * cb, cb)
            pltpu.sync_copy(table_hbm.at[sl_t], cbuf)
            pltpu.sync_copy(cbuf, acc.at[sl_a])

        pltpu.sync_copy(src_hbm.at[pl.ds(s * _NCH_S, _NCH_S)], srcb)
        pltpu.sync_copy(dstp_hbm.at[pl.ds(s * _NCH_S, _NCH_S)], dstb)
        off = c * _NPAD

        @pl.loop(0, _NCH_S)
        def _(i):
            for j in range(_CH // _L):
                sl = pl.ds(j * _L, _L)
                srcb[i, sl] = srcb[i, sl] + off

        plsc.subcore_barrier()
        _ring(table_hbm, acc, srcb, dstb, rows, gsem, ssem, _NCH_S, _NB1)
        plsc.subcore_barrier()

        for i in range(_RPT // cb):
            sl_a = pl.ds(r0 + i * cb, cb)
            sl_o = pl.ds(c * _NPAD + r0 + i * cb, cb)
            pltpu.sync_copy(acc.at[sl_a], cbuf)
            pltpu.sync_copy(cbuf, out_hbm.at[sl_o])

    return k


_RB = 1280


def _dis_of(da):
    deg = da[0, :, 0] + da[1, :, 0] + 1.0
    return lax.rsqrt(deg)


def _b1_body(da_ref, x_ref, out_ref):
    dis = _dis_of(da_ref[...])
    xp = x_ref[...] * dis[:, None]
    out_ref[0] = xp[:, :_HD]
    out_ref[1] = xp[:, _HD:]


def _b2_body(agg_ref, da_ref, w1_ref, b1_ref, w2_ref, out_ref):
    dis = _dis_of(da_ref[...])
    agg = agg_ref[...]
    t = jnp.concatenate([agg[0], agg[1]], axis=1) * dis[:, None]
    h = jnp.dot(t, w1_ref[...], preferred_element_type=jnp.float32)
    h = jnp.maximum(h + b1_ref[...], 0.0)
    g = jnp.dot(h, w2_ref[...], preferred_element_type=jnp.float32)
    out_ref[...] = g * dis[:, None]


def _b3_body(agg_ref, da_ref, b2_ref, out_ref):
    dis = _dis_of(da_ref[...])
    agg = agg_ref[...]
    out_ref[...] = (agg[0] + agg[1]) * dis[:, None] + b2_ref[...]


def kernel(x, edge_index, W1, b1, W2, b2):
    src2d = edge_index[0].reshape(_NW * _NCH_A, _CH)
    dst2d = edge_index[1].reshape(_NW * _NCH_A, _CH)
    xpad = jnp.pad(x, ((0, _NPAD - _N), (0, 0)))
    b1r = b1.reshape(1, _D)
    W2p = jnp.pad(W2, ((0, 0), (0, 16 - _OUT)))
    b2p = jnp.pad(b2, (0, 16 - _OUT)).reshape(1, 16)
    grid = (_NPAD // _RB,)

    degacc, dstp2d = _edge_pass16(False)(src2d, dst2d)
    degacc3 = degacc.reshape(_NC, _NPAD, 16)

    tab1 = pl.pallas_call(
        _b1_body,
        grid=grid,
        in_specs=[
            pl.BlockSpec((_NC, _RB, 16), lambda i: (0, i, 0)),
            pl.BlockSpec((_RB, _D), lambda i: (i, 0)),
        ],
        out_specs=pl.BlockSpec((_NC, _RB, _HD), lambda i: (0, i, 0)),
        out_shape=jax.ShapeDtypeStruct((_NC, _NPAD, _HD), jnp.float32),
    )(degacc3, xpad)

    agg1 = _l1_pass()(src2d, dstp2d, tab1.reshape(_NC * _NPAD, _HD))

    tab2 = pl.pallas_call(
        _b2_body,
        grid=grid,
        in_specs=[
            pl.BlockSpec((_NC, _RB, _HD), lambda i: (0, i, 0)),
            pl.BlockSpec((_NC, _RB, 16), lambda i: (0, i, 0)),
            pl.BlockSpec((_D, _D), lambda i: (0, 0)),
            pl.BlockSpec((1, _D), lambda i: (0, 0)),
            pl.BlockSpec((_D, 16), lambda i: (0, 0)),
        ],
        out_specs=pl.BlockSpec((_RB, 16), lambda i: (i, 0)),
        out_shape=jax.ShapeDtypeStruct((_NPAD, 16), jnp.float32),
    )(agg1.reshape(_NC, _NPAD, _HD), degacc3, W1, b1r, W2p)

    agg2 = _edge_pass16(True)(src2d, dstp2d, tab2)

    outp = pl.pallas_call(
        _b3_body,
        grid=grid,
        in_specs=[
            pl.BlockSpec((_NC, _RB, 16), lambda i: (0, i, 0)),
            pl.BlockSpec((_NC, _RB, 16), lambda i: (0, i, 0)),
            pl.BlockSpec((1, 16), lambda i: (0, 0)),
        ],
        out_specs=pl.BlockSpec((_RB, 16), lambda i: (i, 0)),
        out_shape=jax.ShapeDtypeStruct((_NPAD, 16), jnp.float32),
    )(agg2.reshape(_NC, _NPAD, 16), degacc3, b2p)

    return outp[:_N, :_OUT]


# speedup vs baseline: 1.0282x; 1.0117x over previous
import functools

import jax
import jax.numpy as jnp
from jax import lax
from jax.experimental import pallas as pl
from jax.experimental.pallas import tpu as pltpu
from jax.experimental.pallas import tpu_sc as plsc

_N = 10000
_E = 320000
_D = 128
_HD = _D // 2
_OUT = 4
_NPAD = 10240
_DUMP = _N
_NC = 2
_NS = 16
_L = 16
_NW = _NC * _NS
_CH = 80
_NB = 5
_NB1 = 7
_RPT = _NPAD // _NS

_NCH_A = _E // _NW // _CH
_NCH_S = _E // _NS // _CH

_SC_PARAMS = pltpu.CompilerParams(use_tc_tiling_on_sc=False)
_MESH = dict(core_axis_name="c", subcore_axis_name="s")


def _zero_fill(buf, n_rows, width):
    @pl.loop(0, n_rows)
    def _(i):
        for j in range(width // _L):
            buf[i, pl.ds(j * _L, _L)] = jnp.zeros((_L,), jnp.float32)


def _ring(table_hbm, acc, srcb, dstb, rows, gsem, ssem, n_chunks, nb,
          const_rows=False):
    def g_start(ch, b):
        pltpu.async_copy(table_hbm.at[srcb.at[ch]], rows[b], gsem[b])

    def g_wait(ch, b):
        pltpu.make_async_copy(
            table_hbm.at[srcb.at[ch]], rows[b], gsem[b]).wait()

    def s_start(ch, b):
        rb = rows[0] if const_rows else rows[b]
        pltpu.async_copy(rb, acc.at[dstb.at[ch]], ssem[b], add=True)

    def s_wait(ch, b):
        rb = rows[0] if const_rows else rows[b]
        pltpu.make_async_copy(rb, acc.at[dstb.at[ch]], ssem[b]).wait()

    n_out = n_chunks // nb
    rem = n_chunks - n_out * nb
    if const_rows:
        @pl.loop(0, n_out)
        def _(t):
            base = t * nb
            for b in range(nb):
                s_start(base + b, b)
            for b in range(nb):
                s_wait(base + b, b)
        for b in range(rem):
            s_start(n_out * nb + b, b)
        for b in range(rem):
            s_wait(n_out * nb + b, b)
        return

    for b in range(nb):
        g_start(b, b)

    @pl.loop(0, n_out - 1)
    def _(t):
        base = t * nb
        for b in range(nb):
            g_wait(base + b, b)
            s_start(base + b, b)
        for b in range(nb):
            s_wait(base + b, b)
            g_start(base + nb + b, b)

    tail = (n_out - 1) * nb
    for b in range(nb):
        g_wait(tail + b, b)
        s_start(tail + b, b)
    for b in range(nb):
        s_wait(tail + b, b)
        if b < rem:
            g_start(tail + nb + b, b)
    for b in range(rem):
        g_wait(tail + nb + b, b)
        s_start(tail + nb + b, b)
    for b in range(rem):
        s_wait(tail + nb + b, b)


def _edge_pass16(do_gather, nb=14):
    scratch = (
        [
            pltpu.VMEM((_NCH_A, _CH), jnp.int32),
            pltpu.VMEM((_NCH_A, _CH), jnp.int32),
            pltpu.VMEM((_RPT, 16), jnp.float32),
            pltpu.VMEM_SHARED((_NPAD, 16), jnp.float32),
        ]
        + [pltpu.VMEM((_CH, 16), jnp.float32) for _ in range(nb)]
        + [pltpu.SemaphoreType.DMA for _ in range(2 * nb)]
    )
    if do_gather:
        out_type = jax.ShapeDtypeStruct((_NC * _NPAD, 16), jnp.float32)
    else:
        out_type = (
            jax.ShapeDtypeStruct((_NC * _NPAD, 16), jnp.float32),
            jax.ShapeDtypeStruct((_NW * _NCH_A, _CH), jnp.int32),
        )

    @functools.partial(
        pl.kernel, mesh=plsc.VectorSubcoreMesh(**_MESH),
        out_type=out_type, scratch_types=scratch,
        compiler_params=_SC_PARAMS,
    )
    def k(*refs):
        if do_gather:
            (src_hbm, dstp_hbm, table_hbm, out_hbm), rest = refs[:4], refs[4:]
        else:
            (src_hbm, dst_hbm, out_hbm, dstp_hbm), rest = refs[:4], refs[4:]
            table_hbm = None
        srcb, dstb, cbuf, acc = rest[:4]
        rows = rest[4:4 + nb]
        gsem = rest[4 + nb:4 + 2 * nb]
        ssem = rest[4 + 2 * nb:]

        c = lax.axis_index("c")
        s = lax.axis_index("s")
        w = c * _NS + s
        r0 = s * _RPT

        if do_gather:
            @pl.when(c == 0)
            def _():
                sl = pl.ds(r0, _RPT)
                pltpu.sync_copy(table_hbm.at[sl], cbuf)
                pltpu.sync_copy(cbuf, acc.at[sl])

            @pl.when(c != 0)
            def _():
                _zero_fill(cbuf, _RPT, 16)
                pltpu.sync_copy(cbuf, acc.at[pl.ds(r0, _RPT)])
        else:
            _zero_fill(cbuf, _RPT, 16)
            pltpu.sync_copy(cbuf, acc.at[pl.ds(r0, _RPT)])

        pltpu.sync_copy(src_hbm.at[pl.ds(w * _NCH_A, _NCH_A)], srcb)
        if do_gather:
            pltpu.sync_copy(dstp_hbm.at[pl.ds(w * _NCH_A, _NCH_A)], dstb)
        else:
            pltpu.sync_copy(dst_hbm.at[pl.ds(w * _NCH_A, _NCH_A)], dstb)

            @pl.loop(0, _NCH_A)
            def _(i):
                for j in range(_CH // _L):
                    sl = pl.ds(j * _L, _L)
                    sv = srcb[i, sl]
                    dv = dstb[i, sl]
                    dstb[i, sl] = jnp.where(
                        sv != dv, dv, jnp.full((_L,), _DUMP, jnp.int32))

            pltpu.sync_copy(dstb, dstp_hbm.at[pl.ds(w * _NCH_A, _NCH_A)])

            @pl.loop(0, _CH)
            def _(j):
                rows[0][j, :] = jnp.full((16,), 1.0, jnp.float32)

        plsc.subcore_barrier()
        _ring(table_hbm, acc, srcb, dstb, rows, gsem, ssem, _NCH_A, nb,
              const_rows=not do_gather)
        plsc.subcore_barrier()

        sl_acc = pl.ds(r0, _RPT)
        sl_out = pl.ds(c * _NPAD + r0, _RPT)
        pltpu.sync_copy(acc.at[sl_acc], cbuf)
        pltpu.sync_copy(cbuf, out_hbm.at[sl_out])

    return k


def _l1_pass():
    cb = 128
    scratch = (
        [
            pltpu.VMEM((_NCH_S, _CH), jnp.int32),
            pltpu.VMEM((_NCH_S, _CH), jnp.int32),
            pltpu.VMEM((cb, _HD), jnp.float32),
            pltpu.VMEM_SHARED((_NPAD, _HD), jnp.float32),
        ]
        + [pltpu.VMEM((_CH, _HD), jnp.float32) for _ in range(_NB1)]
        + [pltpu.SemaphoreType.DMA for _ in range(2 * _NB1)]
    )

    @functools.partial(
        pl.kernel, mesh=plsc.VectorSubcoreMesh(**_MESH),
        out_type=jax.ShapeDtypeStruct((_NC * _NPAD, _HD), jnp.float32),
        scratch_types=scratch,
        compiler_params=_SC_PARAMS,
    )
    def k(src_hbm, dstp_hbm, table_hbm, out_hbm, srcb, dstb, cbuf, acc, *rs):
        rows = rs[:_NB1]
        gsem = rs[_NB1:2 * _NB1]
        ssem = rs[2 * _NB1:]

        c = lax.axis_index("c")
        s = lax.axis_index("s")
        r0 = s * _RPT

        for i in range(_RPT // cb):
            sl_t = pl.ds(c * _NPAD + r0 + i * cb, cb)
            sl_a = pl.ds(r0 + i * cb, cb)
            pltpu.sync_copy(table_hbm.at[sl_t], cbuf)
            pltpu.sync_copy(cbuf, acc.at[sl_a])

        pltpu.sync_copy(src_hbm.at[pl.ds(s * _NCH_S, _NCH_S)], srcb)
        pltpu.sync_copy(dstp_hbm.at[pl.ds(s * _NCH_S, _NCH_S)], dstb)
        off = c * _NPAD

        @pl.loop(0, _NCH_S)
        def _(i):
            for j in range(_CH // _L):
                sl = pl.ds(j * _L, _L)
                srcb[i, sl] = srcb[i, sl] + off

        plsc.subcore_barrier()
        _ring(table_hbm, acc, srcb, dstb, rows, gsem, ssem, _NCH_S, _NB1)
        plsc.subcore_barrier()

        for i in range(_RPT // cb):
            sl_a = pl.ds(r0 + i * cb, cb)
            sl_o = pl.ds(c * _NPAD + r0 + i * cb, cb)
            pltpu.sync_copy(acc.at[sl_a], cbuf)
            pltpu.sync_copy(cbuf, out_hbm.at[sl_o])

    return k


_RB = 1280


def _dis_of(da):
    deg = da[0, :, 0] + da[1, :, 0] + 1.0
    return lax.rsqrt(deg)


def _b1_body(da_ref, x_ref, out_ref):
    dis = _dis_of(da_ref[...])
    xp = x_ref[...] * dis[:, None]
    out_ref[0] = xp[:, :_HD]
    out_ref[1] = xp[:, _HD:]


def _b2_body(agg_ref, da_ref, w1_ref, b1_ref, w2_ref, out_ref):
    dis = _dis_of(da_ref[...])
    agg = agg_ref[...]
    t = jnp.concatenate([agg[0], agg[1]], axis=1) * dis[:, None]
    h = jnp.dot(t, w1_ref[...], preferred_element_type=jnp.float32)
    h = jnp.maximum(h + b1_ref[...], 0.0)
    g = jnp.dot(h, w2_ref[...], preferred_element_type=jnp.float32)
    out_ref[...] = g * dis[:, None]


def _b3_body(agg_ref, da_ref, b2_ref, out_ref):
    dis = _dis_of(da_ref[...])
    agg = agg_ref[...]
    out_ref[...] = (agg[0] + agg[1]) * dis[:, None] + b2_ref[...]


def kernel(x, edge_index, W1, b1, W2, b2):
    src2d = edge_index[0].reshape(_NW * _NCH_A, _CH)
    dst2d = edge_index[1].reshape(_NW * _NCH_A, _CH)
    xpad = jnp.pad(x, ((0, _NPAD - _N), (0, 0)))
    b1r = b1.reshape(1, _D)
    W2p = jnp.pad(W2, ((0, 0), (0, 16 - _OUT)))
    b2p = jnp.pad(b2, (0, 16 - _OUT)).reshape(1, 16)
    grid = (_NPAD // _RB,)

    degacc, dstp2d = _edge_pass16(False)(src2d, dst2d)
    degacc3 = degacc.reshape(_NC, _NPAD, 16)

    tab1 = pl.pallas_call(
        _b1_body,
        grid=grid,
        in_specs=[
            pl.BlockSpec((_NC, _RB, 16), lambda i: (0, i, 0)),
            pl.BlockSpec((_RB, _D), lambda i: (i, 0)),
        ],
        out_specs=pl.BlockSpec((_NC, _RB, _HD), lambda i: (0, i, 0)),
        out_shape=jax.ShapeDtypeStruct((_NC, _NPAD, _HD), jnp.float32),
    )(degacc3, xpad)

    agg1 = _l1_pass()(src2d, dstp2d, tab1.reshape(_NC * _NPAD, _HD))

    tab2 = pl.pallas_call(
        _b2_body,
        grid=grid,
        in_specs=[
            pl.BlockSpec((_NC, _RB, _HD), lambda i: (0, i, 0)),
            pl.BlockSpec((_NC, _RB, 16), lambda i: (0, i, 0)),
            pl.BlockSpec((_D, _D), lambda i: (0, 0)),
            pl.BlockSpec((1, _D), lambda i: (0, 0)),
            pl.BlockSpec((_D, 16), lambda i: (0, 0)),
        ],
        out_specs=pl.BlockSpec((_RB, 16), lambda i: (i, 0)),
        out_shape=jax.ShapeDtypeStruct((_NPAD, 16), jnp.float32),
    )(agg1.reshape(_NC, _NPAD, _HD), degacc3, W1, b1r, W2p)

    agg2 = _edge_pass16(True)(src2d, dstp2d, tab2)

    outp = pl.pallas_call(
        _b3_body,
        grid=grid,
        in_specs=[
            pl.BlockSpec((_NC, _RB, 16), lambda i: (0, i, 0)),
            pl.BlockSpec((_NC, _RB, 16), lambda i: (0, i, 0)),
            pl.BlockSpec((1, 16), lambda i: (0, 0)),
        ],
        out_specs=pl.BlockSpec((_RB, 16), lambda i: (i, 0)),
        out_shape=jax.ShapeDtypeStruct((_NPAD, 16), jnp.float32),
    )(agg2.reshape(_NC, _NPAD, 16), degacc3, b2p)

    return outp[:_N, :_OUT]
